# baseline (speedup 1.0000x reference)
import functools

import jax
import jax.numpy as jnp
from jax import lax
from jax.experimental import pallas as pl
from jax.experimental.pallas import tpu as pltpu
from jax.experimental.pallas import tpu_sc as plsc

B, L, D, H = 1024, 200, 64, 8
N = B * L
V = 1000001
NC, NS = 2, 16
NW = NC * NS
UE = 128
NB = B // UE
NUNIT = L * NB
UPW = NUNIT // NW
NQ = D // 16


def _cve_body(v_ref, cm_ref, vid_ref, w1_ref, b1_ref, w2t_ref, corr_ref,
              out_ref, pm_ref):
    for li in range(8):
        v = v_ref[pl.ds(li, 1), :]
        t = jnp.tanh(w1_ref[...] * v + b1_ref[...])
        t = t * cm_ref[pl.ds(li, 1), :]
        ve = lax.dot_general(
            w2t_ref[...], t, (((1,), (0,)), ((), ())),
            preferred_element_type=jnp.float32,
        )
        vid = vid_ref[pl.ds(li, 1), :]
        ve = ve + corr_ref[...] * (vid == V - 1).astype(jnp.float32)
        for b1 in range(NB):
            out_ref[li, :, b1] = ve[:, b1 * UE:(b1 + 1) * UE].reshape(8, 8, UE)
    pm = jnp.minimum(vid_ref[...].astype(jnp.float32), 1.0)
    pm_ref[0] = pm.reshape(8, NB, UE).transpose(1, 0, 2)


def _cve_tc(vT, cmT, vidT, w1c, b1c, W2T, corr):
    return pl.pallas_call(
        _cve_body,
        grid=(L // 8,),
        in_specs=[
            pl.BlockSpec((8, B), lambda l: (l, 0)),
            pl.BlockSpec((8, B), lambda l: (l, 0)),
            pl.BlockSpec((8, B), lambda l: (l, 0)),
            pl.BlockSpec((H, 1), lambda l: (0, 0)),
            pl.BlockSpec((H, 1), lambda l: (0, 0)),
            pl.BlockSpec((D, H), lambda l: (0, 0)),
            pl.BlockSpec((D, 1), lambda l: (0, 0)),
        ],
        out_specs=[
            pl.BlockSpec((8, 8, NB, 8, UE), lambda l: (l, 0, 0, 0, 0)),
            pl.BlockSpec((1, NB, 8, UE), lambda l: (l, 0, 0, 0)),
        ],
        out_shape=[
            jax.ShapeDtypeStruct((L, 8, NB, 8, UE), jnp.float32),
            jax.ShapeDtypeStruct((L // 8, NB, 8, UE), jnp.float32),
        ],
    )(vT, cmT, vidT, w1c, b1c, W2T, corr)


def _sc_gather_add(idx1, ve5, table2):
    mesh = plsc.VectorSubcoreMesh(
        core_axis_name="c", subcore_axis_name="s", num_cores=NC, num_subcores=NS
    )

    @functools.partial(
        pl.kernel,
        out_type=jax.ShapeDtypeStruct((L, 8, NB, 8, 128), jnp.float32),
        mesh=mesh,
        scratch_types=[
            pltpu.VMEM((2, UE), jnp.int32),
            pltpu.VMEM((2, UE), jnp.int32),
            pltpu.VMEM((2, UE + 16), jnp.int32),
            pltpu.VMEM((UE, 128), jnp.float32),
            pltpu.VMEM((UE, 128), jnp.float32),
            pltpu.VMEM((3, 8, 8, UE), jnp.float32),
            pltpu.SemaphoreType.DMA,
            pltpu.SemaphoreType.DMA,
            pltpu.SemaphoreType.DMA,
            pltpu.SemaphoreType.DMA,
        ],
        compiler_params=pltpu.CompilerParams(
            use_tc_tiling_on_sc=True, needs_layout_passes=False
        ),
    )
    def k(idx_hbm, ve_hbm, table_hbm, out_hbm,
          idx_v, p_v, o_v, rows_a, rows_b, t_v,
          sem_i, sem_g, sem_ve, sem_out):
        wid = lax.axis_index("s") * NC + lax.axis_index("c")
        iota16 = lax.iota(jnp.int32, 16)

        def unit_lb(u):
            uid = wid * UPW + u
            l = uid // NB
            return l // 8, l % 8, uid % NB

        def unit_off(u):
            l1, l2, b1 = unit_lb(u)
            return l1 * 8192 + b1 * 1024 + l2 * 128

        def start_idx(u):
            pltpu.async_copy(idx_hbm.at[pl.ds(unit_off(u), UE)],
                             idx_v.at[u % 2], sem_i)

        def wait_idx(u):
            pltpu.make_async_copy(idx_hbm.at[pl.ds(0, UE)],
                                  idx_v.at[u % 2], sem_i).wait()

        def prep_and_gather(u, rows):
            r = u % 2
            for c in range(UE // 16):
                i16 = idx_v[r, pl.ds(16 * c, 16)]
                p_v[r, pl.ds(16 * c, 16)] = jnp.minimum(
                    lax.shift_right_logical(i16, 1), V // 2 - 1)
                o_v[r, pl.ds(16 * c, 16)] = lax.shift_left(
                    lax.bitwise_and(i16, 1), 6)
            pltpu.async_copy(table_hbm.at[p_v.at[r]], rows, sem_g)

        def wait_gather(rows):
            pltpu.make_async_copy(table_hbm.at[pl.ds(0, UE)], rows,
                                  sem_g).wait()

        def start_ve(u):
            l1, l2, b1 = unit_lb(u)
            pltpu.async_copy(ve_hbm.at[l1 * 8 + l2, pl.ds(0, 8), b1],
                             t_v.at[u % 3], sem_ve)

        def wait_ve(u):
            pltpu.make_async_copy(ve_hbm.at[0, pl.ds(0, 8), 0],
                                  t_v.at[u % 3], sem_ve).wait()

        def start_out(u):
            l1, l2, b1 = unit_lb(u)
            pltpu.async_copy(t_v.at[u % 3],
                             out_hbm.at[l1 * 8 + l2, pl.ds(0, 8), b1],
                             sem_out)

        def drain_out(u):
            pltpu.make_async_copy(t_v.at[u % 3],
                                  out_hbm.at[0, pl.ds(0, 8), 0],
                                  sem_out).wait()

        def run_unit(u, rows_cur, rows_nxt):
            slot = u % 3
            wait_gather(rows_cur)
            wait_ve(u)
            @pl.when(u + 1 < UPW)
            def _():
                wait_idx(u + 1)
                prep_and_gather(u + 1, rows_nxt)
            @pl.when(u + 2 < UPW)
            def _():
                start_idx(u + 2)
            @pl.when(u >= 1)
            def _():
                drain_out(u - 1)
            @pl.when(u + 2 < UPW)
            def _():
                start_ve(u + 2)

            ts = t_v.at[slot]
            d1c = [(iota16 + 16 * q) // 8 for q in range(NQ)]
            d2c = [(iota16 + 16 * q) % 8 for q in range(NQ)]
            for e in range(UE):
                off = o_v[u % 2, pl.ds(e, 16)][0]
                espl = jnp.full((16,), e, jnp.int32)
                for q in range(NQ):
                    acc = rows_cur[e, pl.ds(off + 16 * q, 16)]
                    plsc.addupdate_scatter(ts, [d1c[q], d2c[q], espl], acc)
            start_out(u)

        start_idx(0)
        wait_idx(0)
        prep_and_gather(0, rows_a)
        start_idx(1)
        start_ve(0)
        start_ve(1)

        def pair(kk, carry):
            run_unit(2 * kk, rows_a, rows_b)
            run_unit(2 * kk + 1, rows_b, rows_a)
            return carry

        lax.fori_loop(0, UPW // 2, pair, 0, unroll=False)
        drain_out(UPW - 1)

    return k(idx1, ve5, table2)


def kernel(value, var_id, category_mask, W1, b1, W2, emb_table):
    var_id = var_id.astype(jnp.int32)
    vT = value.astype(jnp.float32).T
    cmT = category_mask.astype(jnp.float32).T
    vidT = var_id.T
    idx1 = var_id.reshape(NB, UE, L // 8, 8).transpose(2, 0, 3, 1).reshape(N)
    table2 = emb_table[: V - 1].reshape(V // 2, 128)
    corr = (emb_table[V - 1] - emb_table[V - 3]).reshape(D, 1)
    ve5, pm4 = _cve_tc(vT, cmT, vidT, W1.reshape(H, 1), b1.reshape(H, 1),
                       W2.T, corr)
    out5 = _sc_gather_add(idx1, ve5, emb_table[: V - 1].reshape(V // 2, 128))
    sum_emb = out5.transpose(2, 4, 0, 1, 3).reshape(B, L, D)
    pm = pm4.transpose(1, 3, 0, 2).reshape(B, L)
    return (sum_emb, pm)


# speedup vs baseline: 1.1314x; 1.1314x over previous
import functools

import jax
import jax.numpy as jnp
from jax import lax
from jax.experimental import pallas as pl
from jax.experimental.pallas import tpu as pltpu
from jax.experimental.pallas import tpu_sc as plsc

B, L, D, H = 1024, 200, 64, 8
N = B * L
NC, NS = 2, 16
NW = NC * NS
UE = 128
NB = B // UE
NUNIT = L * NB
UPW = NUNIT // NW
NQ = D // 16
GR = 4
IR = 8


def _cve_body(v_ref, cm_ref, vid_ref, w1_ref, b1_ref, w2t_ref,
              out_ref, pm_ref):
    for li in range(8):
        v = v_ref[pl.ds(li, 1), :]
        t = jnp.tanh(w1_ref[...] * v + b1_ref[...])
        t = t * cm_ref[pl.ds(li, 1), :]
        ve = lax.dot_general(
            w2t_ref[...], t, (((1,), (0,)), ((), ())),
            preferred_element_type=jnp.float32,
        )
        for b1 in range(NB):
            out_ref[li, :, b1] = ve[:, b1 * UE:(b1 + 1) * UE].reshape(8, 8, UE)
    pm = jnp.minimum(vid_ref[...].astype(jnp.float32), 1.0)
    pm_ref[0] = pm.reshape(8, NB, UE).transpose(1, 0, 2)


def _cve_tc(vT, cmT, vidT, w1c, b1c, W2T):
    return pl.pallas_call(
        _cve_body,
        grid=(L // 8,),
        in_specs=[
            pl.BlockSpec((8, B), lambda l: (l, 0)),
            pl.BlockSpec((8, B), lambda l: (l, 0)),
            pl.BlockSpec((8, B), lambda l: (l, 0)),
            pl.BlockSpec((H, 1), lambda l: (0, 0)),
            pl.BlockSpec((H, 1), lambda l: (0, 0)),
            pl.BlockSpec((D, H), lambda l: (0, 0)),
        ],
        out_specs=[
            pl.BlockSpec((8, 8, NB, 8, UE), lambda l: (l, 0, 0, 0, 0)),
            pl.BlockSpec((1, NB, 8, UE), lambda l: (l, 0, 0, 0)),
        ],
        out_shape=[
            jax.ShapeDtypeStruct((L, 8, NB, 8, UE), jnp.float32),
            jax.ShapeDtypeStruct((L // 8, NB, 8, UE), jnp.float32),
        ],
    )(vT, cmT, vidT, w1c, b1c, W2T)


def _sc_gather_t(idx1, table):
    mesh = plsc.VectorSubcoreMesh(
        core_axis_name="c", subcore_axis_name="s", num_cores=NC, num_subcores=NS
    )

    @functools.partial(
        pl.kernel,
        out_type=jax.ShapeDtypeStruct((L, 8, NB, 8, 128), jnp.float32),
        mesh=mesh,
        scratch_types=[
            pltpu.VMEM((IR, UE), jnp.int32),
            pltpu.VMEM((GR, UE, D), jnp.float32),
            pltpu.VMEM((3, 8, 8, UE), jnp.float32),
            pltpu.SemaphoreType.DMA,
            pltpu.SemaphoreType.DMA,
            pltpu.SemaphoreType.DMA,
        ],
        compiler_params=pltpu.CompilerParams(
            use_tc_tiling_on_sc=False, needs_layout_passes=False
        ),
    )
    def k(idx_hbm, table_hbm, out_hbm, idx_v, rows_v, t_v,
          sem_i, sem_g, sem_out):
        wid = lax.axis_index("s") * NC + lax.axis_index("c")
        iota16 = lax.iota(jnp.int32, 16)

        def unit_lb(u):
            uid = wid * UPW + u
            l = uid // NB
            return l // 8, l % 8, uid % NB

        def unit_off(u):
            l1, l2, b1 = unit_lb(u)
            return l1 * 8192 + b1 * 1024 + l2 * 128

        def start_idx(u):
            pltpu.async_copy(idx_hbm.at[pl.ds(unit_off(u), UE)],
                             idx_v.at[u % IR], sem_i)

        def wait_idx(u):
            pltpu.make_async_copy(idx_hbm.at[pl.ds(0, UE)],
                                  idx_v.at[u % IR], sem_i).wait()

        def start_gather(u):
            pltpu.async_copy(table_hbm.at[idx_v.at[u % IR]],
                             rows_v.at[u % GR], sem_g)

        def wait_gather(u):
            pltpu.make_async_copy(table_hbm.at[pl.ds(0, UE)],
                                  rows_v.at[u % GR], sem_g).wait()

        def start_out(u):
            l1, l2, b1 = unit_lb(u)
            pltpu.async_copy(t_v.at[u % 3],
                             out_hbm.at[l1 * 8 + l2, pl.ds(0, 8), b1],
                             sem_out)

        def drain_out(u):
            pltpu.make_async_copy(t_v.at[u % 3],
                                  out_hbm.at[0, pl.ds(0, 8), 0],
                                  sem_out).wait()

        def run_unit(u):
            wait_gather(u)
            @pl.when(u >= 3)
            def _():
                drain_out(u - 3)

            ts = t_v.at[u % 3]
            rs = rows_v.at[u % GR]
            d1c = [(iota16 + 16 * q) // 8 for q in range(NQ)]
            d2c = [(iota16 + 16 * q) % 8 for q in range(NQ)]
            for e in range(UE):
                espl = jnp.full((16,), e, jnp.int32)
                for q in range(NQ):
                    acc = rs[e, pl.ds(16 * q, 16)]
                    plsc.store_scatter(ts, [d1c[q], d2c[q], espl], acc)
            start_out(u)

            @pl.when(u + GR < UPW)
            def _():
                wait_idx(u + GR)
                start_gather(u + GR)
            @pl.when(u + IR - 1 < UPW)
            def _():
                start_idx(u + IR - 1)

        for i in range(IR - 1):
            start_idx(i)
        for i in range(GR):
            wait_idx(i)
            start_gather(i)

        def pair(kk, carry):
            run_unit(2 * kk)
            run_unit(2 * kk + 1)
            return carry

        lax.fori_loop(0, UPW // 2, pair, 0, unroll=False)
        drain_out(UPW - 3)
        drain_out(UPW - 2)
        drain_out(UPW - 1)

    return k(idx1, table)


def kernel(value, var_id, category_mask, W1, b1, W2, emb_table):
    var_id = var_id.astype(jnp.int32)
    vT = value.astype(jnp.float32).T
    cmT = category_mask.astype(jnp.float32).T
    vidT = var_id.T
    idx1 = var_id.reshape(NB, UE, L // 8, 8).transpose(2, 0, 3, 1).reshape(N)
    ve5, pm4 = _cve_tc(vT, cmT, vidT, W1.reshape(H, 1), b1.reshape(H, 1),
                       W2.T)
    g5 = _sc_gather_t(idx1, emb_table)
    sum_emb = (g5 + ve5).transpose(2, 4, 0, 1, 3).reshape(B, L, D)
    pm = pm4.transpose(1, 3, 0, 2).reshape(B, L)
    return (sum_emb, pm)
